# baseline (speedup 1.0000x reference)
import functools
import math

import jax
import jax.numpy as jnp
from jax import lax
from jax.experimental import pallas as pl
from jax.experimental.pallas import tpu as pltpu
from jax.experimental.pallas import tpu_sc as plsc

_NC = 2
_NS = 16
_NW = _NC * _NS
_CH = 128
_LN = 16


def _tc_add(a, b, blk):
    n, d = a.shape

    def body(a_ref, b_ref, o_ref):
        o_ref[...] = a_ref[...] + b_ref[...]

    return pl.pallas_call(
        body,
        grid=(n // blk,),
        in_specs=[pl.BlockSpec((blk, d), lambda i: (i, 0))] * 2,
        out_specs=pl.BlockSpec((blk, d), lambda i: (i, 0)),
        out_shape=jax.ShapeDtypeStruct((n, d), jnp.float32),
    )(a, b)


def _tc_qkv(x, wq, bq_, wk, bk_, wv, bv_, blk):
    n, d = x.shape

    def body(x_ref, wq_ref, bq_ref, wk_ref, bk_ref, wv_ref, bv_ref,
             q_ref, k_ref, v_ref):
        xv = x_ref[...]
        q_ref[...] = jnp.dot(xv, wq_ref[...],
                             preferred_element_type=jnp.float32) + bq_ref[...]
        k_ref[...] = jnp.dot(xv, wk_ref[...],
                             preferred_element_type=jnp.float32) + bk_ref[...]
        v_ref[...] = jnp.dot(xv, wv_ref[...],
                             preferred_element_type=jnp.float32) + bv_ref[...]

    wspec = pl.BlockSpec((d, d), lambda i: (0, 0))
    bspec = pl.BlockSpec((1, d), lambda i: (0, 0))
    xspec = pl.BlockSpec((blk, d), lambda i: (i, 0))
    oshape = jax.ShapeDtypeStruct((n, d), jnp.float32)
    return pl.pallas_call(
        body,
        grid=(n // blk,),
        in_specs=[xspec, wspec, bspec, wspec, bspec, wspec, bspec],
        out_specs=[xspec] * 3,
        out_shape=[oshape] * 3,
    )(x, wq, bq_, wk, bk_, wv, bv_)


def _tc_max(scores):
    nch, h, ch = scores.shape

    def body(s_ref, m_ref, acc):
        i = pl.program_id(0)

        @pl.when(i == 0)
        def _():
            acc[...] = s_ref[0]

        @pl.when(i > 0)
        def _():
            acc[...] = jnp.maximum(acc[...], s_ref[0])

        m_ref[...] = jnp.max(acc[...], axis=1, keepdims=True)

    return pl.pallas_call(
        body,
        grid=(nch,),
        in_specs=[pl.BlockSpec((1, h, ch), lambda i: (i, 0, 0))],
        out_specs=pl.BlockSpec((h, 1), lambda i: (0, 0)),
        out_shape=jax.ShapeDtypeStruct((h, 1), jnp.float32),
        scratch_shapes=[pltpu.VMEM((h, ch), jnp.float32)],
    )(scores)


def _tc_inv_sumexp(scores, m):
    nch, h, ch = scores.shape

    def body(s_ref, m_ref, z_ref, acc):
        i = pl.program_id(0)
        e = jnp.exp(s_ref[0] - m_ref[...])

        @pl.when(i == 0)
        def _():
            acc[...] = e

        @pl.when(i > 0)
        def _():
            acc[...] = acc[...] + e

        z_ref[...] = 1.0 / jnp.sum(acc[...], axis=1, keepdims=True)

    return pl.pallas_call(
        body,
        grid=(nch,),
        in_specs=[pl.BlockSpec((1, h, ch), lambda i: (i, 0, 0)),
                  pl.BlockSpec((h, 1), lambda i: (0, 0))],
        out_specs=pl.BlockSpec((h, 1), lambda i: (0, 0)),
        out_shape=jax.ShapeDtypeStruct((h, 1), jnp.float32),
        scratch_shapes=[pltpu.VMEM((h, ch), jnp.float32)],
    )(scores, m)


def _tc_outproj(o0, o1, res, wo, bo_, g, b, blk):
    n, d = o0.shape

    def body(o0_ref, o1_ref, r_ref, w_ref, b_ref, g_ref, be_ref, out_ref):
        o = o0_ref[...] + o1_ref[...]
        y = jnp.dot(o, w_ref[...], preferred_element_type=jnp.float32)
        y = y + b_ref[...] + r_ref[...]
        mu = jnp.mean(y, axis=-1, keepdims=True)
        var = jnp.mean((y - mu) ** 2, axis=-1, keepdims=True)
        out_ref[...] = (y - mu) / jnp.sqrt(var + 1e-5) * g_ref[...] + be_ref[...]

    xspec = pl.BlockSpec((blk, d), lambda i: (i, 0))
    wspec = pl.BlockSpec((d, d), lambda i: (0, 0))
    bspec = pl.BlockSpec((1, d), lambda i: (0, 0))
    return pl.pallas_call(
        body,
        grid=(n // blk,),
        in_specs=[xspec, xspec, xspec, wspec, bspec, bspec, bspec],
        out_specs=xspec,
        out_shape=jax.ShapeDtypeStruct((n, d), jnp.float32),
    )(o0, o1, res, wo, bo_, g, b)


def _tc_head(x, w, bvec, blk):
    n, d = x.shape

    def body(x_ref, w_ref, b_ref, o_ref):
        o_ref[...] = jnp.dot(x_ref[...], w_ref[...],
                             preferred_element_type=jnp.float32) + b_ref[...]

    return pl.pallas_call(
        body,
        grid=(n // blk,),
        in_specs=[pl.BlockSpec((blk, d), lambda i: (i, 0)),
                  pl.BlockSpec((d, d), lambda i: (0, 0)),
                  pl.BlockSpec((1, d), lambda i: (0, 0))],
        out_specs=pl.BlockSpec((blk, d), lambda i: (i, 0)),
        out_shape=jax.ShapeDtypeStruct((n, d), jnp.float32),
    )(x, w, bvec)


def _tc_head2(a, b, w, bvec, blk):
    n, d = a.shape

    def body(a_ref, b2_ref, w_ref, b_ref, o_ref):
        o_ref[...] = jnp.dot(a_ref[...] + b2_ref[...], w_ref[...],
                             preferred_element_type=jnp.float32) + b_ref[...]

    return pl.pallas_call(
        body,
        grid=(n // blk,),
        in_specs=[pl.BlockSpec((blk, d), lambda i: (i, 0)),
                  pl.BlockSpec((blk, d), lambda i: (i, 0)),
                  pl.BlockSpec((d, d), lambda i: (0, 0)),
                  pl.BlockSpec((1, d), lambda i: (0, 0))],
        out_specs=pl.BlockSpec((blk, d), lambda i: (i, 0)),
        out_shape=jax.ShapeDtypeStruct((n, d), jnp.float32),
    )(a, b, w, bvec)


def _sc_scores(q, k, dst, src):
    n, d = q.shape
    h = d // _LN
    e = dst.shape[0]
    nch = e // _CH
    steps = (nch + _NW - 1) // _NW
    steps += steps % 2
    scale = 1.0 / math.sqrt(_LN)
    mesh = plsc.VectorSubcoreMesh(core_axis_name="c", subcore_axis_name="s")

    @functools.partial(
        pl.kernel,
        mesh=mesh,
        compiler_params=pltpu.CompilerParams(needs_layout_passes=False),
        out_type=jax.ShapeDtypeStruct((nch, h, _CH), jnp.float32),
        scratch_types=[
            pltpu.VMEM((_CH,), jnp.int32),
            pltpu.VMEM((_CH,), jnp.int32),
            pltpu.VMEM((_CH,), jnp.int32),
            pltpu.VMEM((_CH,), jnp.int32),
            pltpu.VMEM((_CH, d), jnp.float32),
            pltpu.VMEM((_CH, d), jnp.float32),
            pltpu.VMEM((_CH, d), jnp.float32),
            pltpu.VMEM((_CH, d), jnp.float32),
            pltpu.VMEM((h, _CH), jnp.float32),
            pltpu.SemaphoreType.DMA,
            pltpu.SemaphoreType.DMA,
        ],
    )
    def run(q_hbm, k_hbm, dst_hbm, src_hbm, out_hbm, idx_d0, idx_s0, idx_d1,
            idx_s1, qr0, kr0, qr1, kr1, sb, sem0, sem1):
        wid = lax.axis_index("s") * _NC + lax.axis_index("c")

        def issue(c, idx_d, idx_s, qr, kr, sem):
            @pl.when(c < nch)
            def _():
                base = c * _CH
                pltpu.sync_copy(dst_hbm.at[pl.ds(base, _CH)], idx_d)
                pltpu.sync_copy(src_hbm.at[pl.ds(base, _CH)], idx_s)
                pltpu.async_copy(q_hbm.at[idx_d], qr, sem)
                pltpu.async_copy(k_hbm.at[idx_s], kr, sem)

        def compute(c, idx_d, idx_s, qr, kr, sem):
            @pl.when(c < nch)
            def _():
                pltpu.make_async_copy(q_hbm.at[idx_d], qr, sem).wait()
                pltpu.make_async_copy(k_hbm.at[idx_s], kr, sem).wait()

                def group(gi, cc):
                    rows = gi * _LN + lax.iota(jnp.int32, _LN)
                    for hh in range(h):
                        acc = jnp.zeros((_LN,), jnp.float32)
                        for j in range(_LN):
                            col = jnp.full((_LN,), hh * _LN + j, jnp.int32)
                            qv = plsc.load_gather(qr, [rows, col])
                            kv = plsc.load_gather(kr, [rows, col])
                            acc = acc + qv * kv
                        sb[hh, pl.ds(gi * _LN, _LN)] = acc * scale
                    return cc

                lax.fori_loop(0, _CH // _LN, group, 0)
                pltpu.sync_copy(sb, out_hbm.at[c])

        issue(wid, idx_d0, idx_s0, qr0, kr0, sem0)

        def pair(p, carry):
            s0 = p * 2
            issue(wid + (s0 + 1) * _NW, idx_d1, idx_s1, qr1, kr1, sem1)
            compute(wid + s0 * _NW, idx_d0, idx_s0, qr0, kr0, sem0)
            issue(wid + (s0 + 2) * _NW, idx_d0, idx_s0, qr0, kr0, sem0)
            compute(wid + (s0 + 1) * _NW, idx_d1, idx_s1, qr1, kr1, sem1)
            return carry

        lax.fori_loop(0, steps // 2, pair, 0)

    return run(q, k, dst, src)


def _sc_aggregate(vv, scores, m16, iz16, dst, src):
    n, d = vv.shape
    h = d // _LN
    e = dst.shape[0]
    nch = e // _CH
    steps = (nch + _NW - 1) // _NW
    steps += steps % 2
    npad = -(-n // (_NS * _CH)) * (_NS * _CH)
    rows_sub = npad // _NS
    mesh = plsc.VectorSubcoreMesh(core_axis_name="c", subcore_axis_name="s")

    @functools.partial(
        pl.kernel,
        mesh=mesh,
        compiler_params=pltpu.CompilerParams(needs_layout_passes=False),
        out_type=jax.ShapeDtypeStruct((_NC, npad, d), jnp.float32),
        scratch_types=[
            pltpu.VMEM((_CH,), jnp.int32),
            pltpu.VMEM((_CH,), jnp.int32),
            pltpu.VMEM((_CH,), jnp.int32),
            pltpu.VMEM((_CH,), jnp.int32),
            pltpu.VMEM((_CH, d), jnp.float32),
            pltpu.VMEM((_CH, d), jnp.float32),
            pltpu.VMEM((h, _CH), jnp.float32),
            pltpu.VMEM((h, _CH), jnp.float32),
            pltpu.VMEM((_LN,), jnp.float32),
            pltpu.VMEM((_LN,), jnp.float32),
            pltpu.VMEM_SHARED((npad, d), jnp.float32),
            pltpu.SemaphoreType.DMA,
            pltpu.SemaphoreType.DMA,
        ],
    )
    def run(v_hbm, s_hbm, m_hbm, z_hbm, dst_hbm, src_hbm, out_hbm,
            idx_d0, idx_s0, idx_d1, idx_s1, vr0, vr1, sb0, sb1, mb, zb, acc,
            sem0, sem1):
        cid = lax.axis_index("c")
        sid = lax.axis_index("s")
        wid = sid * _NC + cid

        def zrow(r, cc):
            for hh in range(h):
                vr0[r, pl.ds(hh * _LN, _LN)] = jnp.zeros((_LN,), jnp.float32)
            return cc

        lax.fori_loop(0, _CH, zrow, 0)
        for j in range(rows_sub // _CH):
            pltpu.sync_copy(vr0.at[pl.ds(0, _CH)],
                            acc.at[pl.ds(sid * rows_sub + j * _CH, _CH)])
        plsc.subcore_barrier()
        pltpu.sync_copy(m_hbm, mb)
        pltpu.sync_copy(z_hbm, zb)

        def issue(c, idx_d, idx_s, vr, sb, sem):
            @pl.when(c < nch)
            def _():
                base = c * _CH
                pltpu.sync_copy(dst_hbm.at[pl.ds(base, _CH)], idx_d)
                pltpu.sync_copy(src_hbm.at[pl.ds(base, _CH)], idx_s)
                pltpu.async_copy(v_hbm.at[idx_s], vr, sem)
                pltpu.async_copy(s_hbm.at[c], sb, sem)

        def compute(c, idx_d, idx_s, vr, sb, sem):
            @pl.when(c < nch)
            def _():
                pltpu.make_async_copy(v_hbm.at[idx_s], vr, sem).wait()
                pltpu.make_async_copy(s_hbm.at[c], sb, sem).wait()
                mv = mb[...]
                zv = zb[...]

                def group(gi, cc):
                    rows = gi * _LN + lax.iota(jnp.int32, _LN)
                    gsl = pl.ds(gi * _LN, _LN)
                    for hh in range(h):
                        wv = jnp.exp(sb[hh, gsl] - mv[hh]) * zv[hh]
                        for j in range(_LN):
                            col = jnp.full((_LN,), hh * _LN + j, jnp.int32)
                            x = plsc.load_gather(vr, [rows, col])
                            plsc.store_scatter(vr, [rows, col], x * wv)
                    return cc

                lax.fori_loop(0, _CH // _LN, group, 0)
                pltpu.sync_copy(vr, acc.at[idx_d], add=True)

        issue(wid, idx_d0, idx_s0, vr0, sb0, sem0)

        def pair(p, carry):
            s0 = p * 2
            issue(wid + (s0 + 1) * _NW, idx_d1, idx_s1, vr1, sb1, sem1)
            compute(wid + s0 * _NW, idx_d0, idx_s0, vr0, sb0, sem0)
            issue(wid + (s0 + 2) * _NW, idx_d0, idx_s0, vr0, sb0, sem0)
            compute(wid + (s0 + 1) * _NW, idx_d1, idx_s1, vr1, sb1, sem1)
            return carry

        lax.fori_loop(0, steps // 2, pair, 0)
        plsc.subcore_barrier()
        for j in range(rows_sub // _CH):
            r0 = sid * rows_sub + j * _CH
            pltpu.sync_copy(acc.at[pl.ds(r0, _CH)],
                            out_hbm.at[cid, pl.ds(r0, _CH)])

    return run(vv, scores, m16, iz16, dst, src)


def kernel(vertex_ids, edge_ids, triangle_ids, edge_index, vertex_embed,
           edge_embed, triangle_embed, vertex_pos, edge_pos, triangle_pos,
           Wq, bq, Wk, bk, Wv, bv, Wo, bo, ln_g, ln_b, Wvh, bvh, Weh, beh,
           Wth, bth):
    n, d = vertex_embed.shape
    h = d // _LN
    nl = Wq.shape[0]
    src = edge_index[0]
    dst = edge_index[1]

    v = _tc_add(vertex_embed, vertex_pos, 1000)
    for i in range(nl):
        q, k, vv = _tc_qkv(v, Wq[i], bq[i].reshape(1, d), Wk[i],
                           bk[i].reshape(1, d), Wv[i], bv[i].reshape(1, d),
                           1000)
        scores = _sc_scores(q, k, dst, src)
        m = _tc_max(scores)
        iz = _tc_inv_sumexp(scores, m)
        m16 = jnp.pad(m.reshape(h), (0, _LN - h))
        iz16 = jnp.pad(iz.reshape(h), (0, _LN - h))
        out2 = _sc_aggregate(vv, scores, m16, iz16, dst, src)
        v = _tc_outproj(out2[0, :n], out2[1, :n], v, Wo[i], bo[i].reshape(1, d),
                        ln_g[i].reshape(1, d), ln_b[i].reshape(1, d), 1000)

    v_out = _tc_head(v, Wvh, bvh.reshape(1, d), 1000)
    e_out = _tc_head2(edge_embed, edge_pos, Weh, beh.reshape(1, d), 2000)
    t_out = _tc_head2(triangle_embed, triangle_pos, Wth, bth.reshape(1, d),
                      2000)
    return (v_out, e_out, t_out)


# speedup vs baseline: 1.7504x; 1.7504x over previous
import functools
import math

import jax
import jax.numpy as jnp
from jax import lax
from jax.experimental import pallas as pl
from jax.experimental.pallas import tpu as pltpu
from jax.experimental.pallas import tpu_sc as plsc

_NC = 2
_NS = 16
_NW = _NC * _NS
_CH = 128
_LN = 16


def _tc_add(a, b, blk):
    n, d = a.shape

    def body(a_ref, b_ref, o_ref):
        o_ref[...] = a_ref[...] + b_ref[...]

    return pl.pallas_call(
        body,
        grid=(n // blk,),
        in_specs=[pl.BlockSpec((blk, d), lambda i: (i, 0))] * 2,
        out_specs=pl.BlockSpec((blk, d), lambda i: (i, 0)),
        out_shape=jax.ShapeDtypeStruct((n, d), jnp.float32),
    )(a, b)


def _tc_qkv(x, wq, bq_, wk, bk_, wv, bv_, blk):
    n, d = x.shape

    def body(x_ref, wq_ref, bq_ref, wk_ref, bk_ref, wv_ref, bv_ref,
             q_ref, k_ref, v_ref):
        xv = x_ref[...]
        q_ref[...] = jnp.dot(xv, wq_ref[...],
                             preferred_element_type=jnp.float32) + bq_ref[...]
        k_ref[...] = jnp.dot(xv, wk_ref[...],
                             preferred_element_type=jnp.float32) + bk_ref[...]
        v_ref[...] = jnp.dot(xv, wv_ref[...],
                             preferred_element_type=jnp.float32) + bv_ref[...]

    wspec = pl.BlockSpec((d, d), lambda i: (0, 0))
    bspec = pl.BlockSpec((1, d), lambda i: (0, 0))
    xspec = pl.BlockSpec((blk, d), lambda i: (i, 0))
    oshape = jax.ShapeDtypeStruct((n, d), jnp.float32)
    return pl.pallas_call(
        body,
        grid=(n // blk,),
        in_specs=[xspec, wspec, bspec, wspec, bspec, wspec, bspec],
        out_specs=[xspec] * 3,
        out_shape=[oshape] * 3,
    )(x, wq, bq_, wk, bk_, wv, bv_)


def _tc_max(scores):
    nch, h, ch = scores.shape

    def body(s_ref, m_ref, acc):
        i = pl.program_id(0)

        @pl.when(i == 0)
        def _():
            acc[...] = s_ref[0]

        @pl.when(i > 0)
        def _():
            acc[...] = jnp.maximum(acc[...], s_ref[0])

        m_ref[...] = jnp.max(acc[...], axis=1, keepdims=True)

    return pl.pallas_call(
        body,
        grid=(nch,),
        in_specs=[pl.BlockSpec((1, h, ch), lambda i: (i, 0, 0))],
        out_specs=pl.BlockSpec((h, 1), lambda i: (0, 0)),
        out_shape=jax.ShapeDtypeStruct((h, 1), jnp.float32),
        scratch_shapes=[pltpu.VMEM((h, ch), jnp.float32)],
    )(scores)


def _tc_inv_sumexp(scores, m):
    nch, h, ch = scores.shape

    def body(s_ref, m_ref, z_ref, acc):
        i = pl.program_id(0)
        e = jnp.exp(s_ref[0] - m_ref[...])

        @pl.when(i == 0)
        def _():
            acc[...] = e

        @pl.when(i > 0)
        def _():
            acc[...] = acc[...] + e

        z_ref[...] = 1.0 / jnp.sum(acc[...], axis=1, keepdims=True)

    return pl.pallas_call(
        body,
        grid=(nch,),
        in_specs=[pl.BlockSpec((1, h, ch), lambda i: (i, 0, 0)),
                  pl.BlockSpec((h, 1), lambda i: (0, 0))],
        out_specs=pl.BlockSpec((h, 1), lambda i: (0, 0)),
        out_shape=jax.ShapeDtypeStruct((h, 1), jnp.float32),
        scratch_shapes=[pltpu.VMEM((h, ch), jnp.float32)],
    )(scores, m)


def _tc_outproj(o0, o1, res, wo, bo_, g, b, blk):
    n, d = o0.shape

    def body(o0_ref, o1_ref, r_ref, w_ref, b_ref, g_ref, be_ref, out_ref):
        o = o0_ref[...] + o1_ref[...]
        y = jnp.dot(o, w_ref[...], preferred_element_type=jnp.float32)
        y = y + b_ref[...] + r_ref[...]
        mu = jnp.mean(y, axis=-1, keepdims=True)
        var = jnp.mean((y - mu) ** 2, axis=-1, keepdims=True)
        out_ref[...] = (y - mu) / jnp.sqrt(var + 1e-5) * g_ref[...] + be_ref[...]

    xspec = pl.BlockSpec((blk, d), lambda i: (i, 0))
    wspec = pl.BlockSpec((d, d), lambda i: (0, 0))
    bspec = pl.BlockSpec((1, d), lambda i: (0, 0))
    return pl.pallas_call(
        body,
        grid=(n // blk,),
        in_specs=[xspec, xspec, xspec, wspec, bspec, bspec, bspec],
        out_specs=xspec,
        out_shape=jax.ShapeDtypeStruct((n, d), jnp.float32),
    )(o0, o1, res, wo, bo_, g, b)


def _tc_head(x, w, bvec, blk):
    n, d = x.shape

    def body(x_ref, w_ref, b_ref, o_ref):
        o_ref[...] = jnp.dot(x_ref[...], w_ref[...],
                             preferred_element_type=jnp.float32) + b_ref[...]

    return pl.pallas_call(
        body,
        grid=(n // blk,),
        in_specs=[pl.BlockSpec((blk, d), lambda i: (i, 0)),
                  pl.BlockSpec((d, d), lambda i: (0, 0)),
                  pl.BlockSpec((1, d), lambda i: (0, 0))],
        out_specs=pl.BlockSpec((blk, d), lambda i: (i, 0)),
        out_shape=jax.ShapeDtypeStruct((n, d), jnp.float32),
    )(x, w, bvec)


def _tc_head2(a, b, w, bvec, blk):
    n, d = a.shape

    def body(a_ref, b2_ref, w_ref, b_ref, o_ref):
        o_ref[...] = jnp.dot(a_ref[...] + b2_ref[...], w_ref[...],
                             preferred_element_type=jnp.float32) + b_ref[...]

    return pl.pallas_call(
        body,
        grid=(n // blk,),
        in_specs=[pl.BlockSpec((blk, d), lambda i: (i, 0)),
                  pl.BlockSpec((blk, d), lambda i: (i, 0)),
                  pl.BlockSpec((d, d), lambda i: (0, 0)),
                  pl.BlockSpec((1, d), lambda i: (0, 0))],
        out_specs=pl.BlockSpec((blk, d), lambda i: (i, 0)),
        out_shape=jax.ShapeDtypeStruct((n, d), jnp.float32),
    )(a, b, w, bvec)


def _sc_scores(q, k, dst, src):
    n, d = q.shape
    h = d // _LN
    e = dst.shape[0]
    nch = e // _CH
    steps = (nch + _NW - 1) // _NW
    steps += steps % 2
    scale = 1.0 / math.sqrt(_LN)
    mesh = plsc.VectorSubcoreMesh(core_axis_name="c", subcore_axis_name="s")

    @functools.partial(
        pl.kernel,
        mesh=mesh,
        compiler_params=pltpu.CompilerParams(needs_layout_passes=False),
        out_type=jax.ShapeDtypeStruct((nch, h, _CH), jnp.float32),
        scratch_types=[
            pltpu.VMEM((_CH,), jnp.int32),
            pltpu.VMEM((_CH,), jnp.int32),
            pltpu.VMEM((_CH,), jnp.int32),
            pltpu.VMEM((_CH,), jnp.int32),
            pltpu.VMEM((_CH, d), jnp.float32),
            pltpu.VMEM((_CH, d), jnp.float32),
            pltpu.VMEM((_CH, d), jnp.float32),
            pltpu.VMEM((_CH, d), jnp.float32),
            pltpu.VMEM((h, _CH), jnp.float32),
            pltpu.SemaphoreType.DMA,
            pltpu.SemaphoreType.DMA,
        ],
    )
    def run(q_hbm, k_hbm, dst_hbm, src_hbm, out_hbm, idx_d0, idx_s0, idx_d1,
            idx_s1, qr0, kr0, qr1, kr1, sb, sem0, sem1):
        wid = lax.axis_index("s") * _NC + lax.axis_index("c")

        def issue(c, idx_d, idx_s, qr, kr, sem):
            @pl.when(c < nch)
            def _():
                base = c * _CH
                pltpu.sync_copy(dst_hbm.at[pl.ds(base, _CH)], idx_d)
                pltpu.sync_copy(src_hbm.at[pl.ds(base, _CH)], idx_s)
                pltpu.async_copy(q_hbm.at[idx_d], qr, sem)
                pltpu.async_copy(k_hbm.at[idx_s], kr, sem)

        def compute(c, idx_d, idx_s, qr, kr, sem):
            @pl.when(c < nch)
            def _():
                pltpu.make_async_copy(q_hbm.at[idx_d], qr, sem).wait()
                pltpu.make_async_copy(k_hbm.at[idx_s], kr, sem).wait()

                def group(gi, cc):
                    rot = lax.iota(jnp.int32, _LN)
                    rows = gi * _LN + rot
                    for hh in range(h):
                        acc = jnp.zeros((_LN,), jnp.float32)
                        for j in range(_LN):
                            col = hh * _LN + ((rot + j) & (_LN - 1))
                            qv = plsc.load_gather(qr, [rows, col])
                            kv = plsc.load_gather(kr, [rows, col])
                            acc = acc + qv * kv
                        sb[hh, pl.ds(gi * _LN, _LN)] = acc * scale
                    return cc

                lax.fori_loop(0, _CH // _LN, group, 0)
                pltpu.sync_copy(sb, out_hbm.at[c])

        issue(wid, idx_d0, idx_s0, qr0, kr0, sem0)

        def pair(p, carry):
            s0 = p * 2
            issue(wid + (s0 + 1) * _NW, idx_d1, idx_s1, qr1, kr1, sem1)
            compute(wid + s0 * _NW, idx_d0, idx_s0, qr0, kr0, sem0)
            issue(wid + (s0 + 2) * _NW, idx_d0, idx_s0, qr0, kr0, sem0)
            compute(wid + (s0 + 1) * _NW, idx_d1, idx_s1, qr1, kr1, sem1)
            return carry

        lax.fori_loop(0, steps // 2, pair, 0)

    return run(q, k, dst, src)


def _sc_aggregate(vv, scores, m16, iz16, dst, src):
    n, d = vv.shape
    h = d // _LN
    e = dst.shape[0]
    nch = e // _CH
    steps = (nch + _NW - 1) // _NW
    steps += steps % 2
    npad = -(-n // (_NS * _CH)) * (_NS * _CH)
    rows_sub = npad // _NS
    mesh = plsc.VectorSubcoreMesh(core_axis_name="c", subcore_axis_name="s")

    @functools.partial(
        pl.kernel,
        mesh=mesh,
        compiler_params=pltpu.CompilerParams(needs_layout_passes=False),
        out_type=jax.ShapeDtypeStruct((_NC, npad, d), jnp.float32),
        scratch_types=[
            pltpu.VMEM((_CH,), jnp.int32),
            pltpu.VMEM((_CH,), jnp.int32),
            pltpu.VMEM((_CH,), jnp.int32),
            pltpu.VMEM((_CH,), jnp.int32),
            pltpu.VMEM((_CH, d), jnp.float32),
            pltpu.VMEM((_CH, d), jnp.float32),
            pltpu.VMEM((h, _CH), jnp.float32),
            pltpu.VMEM((h, _CH), jnp.float32),
            pltpu.VMEM((_LN,), jnp.float32),
            pltpu.VMEM((_LN,), jnp.float32),
            pltpu.VMEM_SHARED((npad, d), jnp.float32),
            pltpu.SemaphoreType.DMA,
            pltpu.SemaphoreType.DMA,
        ],
    )
    def run(v_hbm, s_hbm, m_hbm, z_hbm, dst_hbm, src_hbm, out_hbm,
            idx_d0, idx_s0, idx_d1, idx_s1, vr0, vr1, sb0, sb1, mb, zb, acc,
            sem0, sem1):
        cid = lax.axis_index("c")
        sid = lax.axis_index("s")
        wid = sid * _NC + cid

        def zrow(r, cc):
            for hh in range(h):
                vr0[r, pl.ds(hh * _LN, _LN)] = jnp.zeros((_LN,), jnp.float32)
            return cc

        lax.fori_loop(0, _CH, zrow, 0)
        for j in range(rows_sub // _CH):
            pltpu.sync_copy(vr0.at[pl.ds(0, _CH)],
                            acc.at[pl.ds(sid * rows_sub + j * _CH, _CH)])
        plsc.subcore_barrier()
        pltpu.sync_copy(m_hbm, mb)
        pltpu.sync_copy(z_hbm, zb)

        def issue(c, idx_d, idx_s, vr, sb, sem):
            @pl.when(c < nch)
            def _():
                base = c * _CH
                pltpu.sync_copy(dst_hbm.at[pl.ds(base, _CH)], idx_d)
                pltpu.sync_copy(src_hbm.at[pl.ds(base, _CH)], idx_s)
                pltpu.async_copy(v_hbm.at[idx_s], vr, sem)
                pltpu.async_copy(s_hbm.at[c], sb, sem)

        def compute(c, idx_d, idx_s, vr, sb, sem):
            @pl.when(c < nch)
            def _():
                pltpu.make_async_copy(v_hbm.at[idx_s], vr, sem).wait()
                pltpu.make_async_copy(s_hbm.at[c], sb, sem).wait()
                mv = mb[...]
                zv = zb[...]

                def group(gi, cc):
                    rot = lax.iota(jnp.int32, _LN)
                    rows = gi * _LN + rot
                    gsl = pl.ds(gi * _LN, _LN)
                    for hh in range(h):
                        wv = jnp.exp(sb[hh, gsl] - mv[hh]) * zv[hh]
                        for j in range(_LN):
                            col = hh * _LN + ((rot + j) & (_LN - 1))
                            x = plsc.load_gather(vr, [rows, col])
                            plsc.store_scatter(vr, [rows, col], x * wv)
                    return cc

                lax.fori_loop(0, _CH // _LN, group, 0)
                pltpu.sync_copy(vr, acc.at[idx_d], add=True)

        issue(wid, idx_d0, idx_s0, vr0, sb0, sem0)

        def pair(p, carry):
            s0 = p * 2
            issue(wid + (s0 + 1) * _NW, idx_d1, idx_s1, vr1, sb1, sem1)
            compute(wid + s0 * _NW, idx_d0, idx_s0, vr0, sb0, sem0)
            issue(wid + (s0 + 2) * _NW, idx_d0, idx_s0, vr0, sb0, sem0)
            compute(wid + (s0 + 1) * _NW, idx_d1, idx_s1, vr1, sb1, sem1)
            return carry

        lax.fori_loop(0, steps // 2, pair, 0)
        plsc.subcore_barrier()
        for j in range(rows_sub // _CH):
            r0 = sid * rows_sub + j * _CH
            pltpu.sync_copy(acc.at[pl.ds(r0, _CH)],
                            out_hbm.at[cid, pl.ds(r0, _CH)])

    return run(vv, scores, m16, iz16, dst, src)


def kernel(vertex_ids, edge_ids, triangle_ids, edge_index, vertex_embed,
           edge_embed, triangle_embed, vertex_pos, edge_pos, triangle_pos,
           Wq, bq, Wk, bk, Wv, bv, Wo, bo, ln_g, ln_b, Wvh, bvh, Weh, beh,
           Wth, bth):
    n, d = vertex_embed.shape
    h = d // _LN
    nl = Wq.shape[0]
    src = edge_index[0]
    dst = edge_index[1]

    v = _tc_add(vertex_embed, vertex_pos, 1000)
    for i in range(nl):
        q, k, vv = _tc_qkv(v, Wq[i], bq[i].reshape(1, d), Wk[i],
                           bk[i].reshape(1, d), Wv[i], bv[i].reshape(1, d),
                           1000)
        scores = _sc_scores(q, k, dst, src)
        m = _tc_max(scores)
        iz = _tc_inv_sumexp(scores, m)
        m16 = jnp.pad(m.reshape(h), (0, _LN - h))
        iz16 = jnp.pad(iz.reshape(h), (0, _LN - h))
        out2 = _sc_aggregate(vv, scores, m16, iz16, dst, src)
        v = _tc_outproj(out2[0, :n], out2[1, :n], v, Wo[i], bo[i].reshape(1, d),
                        ln_g[i].reshape(1, d), ln_b[i].reshape(1, d), 1000)

    v_out = _tc_head(v, Wvh, bvh.reshape(1, d), 1000)
    e_out = _tc_head2(edge_embed, edge_pos, Weh, beh.reshape(1, d), 2000)
    t_out = _tc_head2(triangle_embed, triangle_pos, Wth, bth.reshape(1, d),
                      2000)
    return (v_out, e_out, t_out)


# speedup vs baseline: 1.9679x; 1.1243x over previous
import functools
import math

import jax
import jax.numpy as jnp
from jax import lax
from jax.experimental import pallas as pl
from jax.experimental.pallas import tpu as pltpu
from jax.experimental.pallas import tpu_sc as plsc

_NC = 2
_NS = 16
_NW = _NC * _NS
_CH = 128
_LN = 16


def _tc_add(a, b, blk):
    n, d = a.shape

    def body(a_ref, b_ref, o_ref):
        o_ref[...] = a_ref[...] + b_ref[...]

    return pl.pallas_call(
        body,
        grid=(n // blk,),
        in_specs=[pl.BlockSpec((blk, d), lambda i: (i, 0))] * 2,
        out_specs=pl.BlockSpec((blk, d), lambda i: (i, 0)),
        out_shape=jax.ShapeDtypeStruct((n, d), jnp.float32),
    )(a, b)


def _tc_qkv(x, wq, bq_, wk, bk_, wv, bv_, blk):
    n, d = x.shape

    def body(x_ref, wq_ref, bq_ref, wk_ref, bk_ref, wv_ref, bv_ref,
             q_ref, k_ref, v_ref):
        xv = x_ref[...]
        q_ref[...] = jnp.dot(xv, wq_ref[...],
                             preferred_element_type=jnp.float32) + bq_ref[...]
        k_ref[...] = jnp.dot(xv, wk_ref[...],
                             preferred_element_type=jnp.float32) + bk_ref[...]
        v_ref[...] = jnp.dot(xv, wv_ref[...],
                             preferred_element_type=jnp.float32) + bv_ref[...]

    wspec = pl.BlockSpec((d, d), lambda i: (0, 0))
    bspec = pl.BlockSpec((1, d), lambda i: (0, 0))
    xspec = pl.BlockSpec((blk, d), lambda i: (i, 0))
    oshape = jax.ShapeDtypeStruct((n, d), jnp.float32)
    return pl.pallas_call(
        body,
        grid=(n // blk,),
        in_specs=[xspec, wspec, bspec, wspec, bspec, wspec, bspec],
        out_specs=[xspec] * 3,
        out_shape=[oshape] * 3,
    )(x, wq, bq_, wk, bk_, wv, bv_)


def _tc_max(scores):
    nch, h, ch = scores.shape

    def body(s_ref, m_ref, acc):
        i = pl.program_id(0)

        @pl.when(i == 0)
        def _():
            acc[...] = s_ref[0]

        @pl.when(i > 0)
        def _():
            acc[...] = jnp.maximum(acc[...], s_ref[0])

        m_ref[...] = jnp.max(acc[...], axis=1, keepdims=True)

    return pl.pallas_call(
        body,
        grid=(nch,),
        in_specs=[pl.BlockSpec((1, h, ch), lambda i: (i, 0, 0))],
        out_specs=pl.BlockSpec((h, 1), lambda i: (0, 0)),
        out_shape=jax.ShapeDtypeStruct((h, 1), jnp.float32),
        scratch_shapes=[pltpu.VMEM((h, ch), jnp.float32)],
    )(scores)


def _tc_inv_sumexp(scores, m):
    nch, h, ch = scores.shape

    def body(s_ref, m_ref, z_ref, acc):
        i = pl.program_id(0)
        e = jnp.exp(s_ref[0] - m_ref[...])

        @pl.when(i == 0)
        def _():
            acc[...] = e

        @pl.when(i > 0)
        def _():
            acc[...] = acc[...] + e

        z_ref[...] = 1.0 / jnp.sum(acc[...], axis=1, keepdims=True)

    return pl.pallas_call(
        body,
        grid=(nch,),
        in_specs=[pl.BlockSpec((1, h, ch), lambda i: (i, 0, 0)),
                  pl.BlockSpec((h, 1), lambda i: (0, 0))],
        out_specs=pl.BlockSpec((h, 1), lambda i: (0, 0)),
        out_shape=jax.ShapeDtypeStruct((h, 1), jnp.float32),
        scratch_shapes=[pltpu.VMEM((h, ch), jnp.float32)],
    )(scores, m)


def _tc_outproj(o0, o1, res, wo, bo_, g, b, blk):
    n, d = o0.shape

    def body(o0_ref, o1_ref, r_ref, w_ref, b_ref, g_ref, be_ref, out_ref):
        o = o0_ref[...] + o1_ref[...]
        y = jnp.dot(o, w_ref[...], preferred_element_type=jnp.float32)
        y = y + b_ref[...] + r_ref[...]
        mu = jnp.mean(y, axis=-1, keepdims=True)
        var = jnp.mean((y - mu) ** 2, axis=-1, keepdims=True)
        out_ref[...] = (y - mu) / jnp.sqrt(var + 1e-5) * g_ref[...] + be_ref[...]

    xspec = pl.BlockSpec((blk, d), lambda i: (i, 0))
    wspec = pl.BlockSpec((d, d), lambda i: (0, 0))
    bspec = pl.BlockSpec((1, d), lambda i: (0, 0))
    return pl.pallas_call(
        body,
        grid=(n // blk,),
        in_specs=[xspec, xspec, xspec, wspec, bspec, bspec, bspec],
        out_specs=xspec,
        out_shape=jax.ShapeDtypeStruct((n, d), jnp.float32),
    )(o0, o1, res, wo, bo_, g, b)


def _tc_head(x, w, bvec, blk):
    n, d = x.shape

    def body(x_ref, w_ref, b_ref, o_ref):
        o_ref[...] = jnp.dot(x_ref[...], w_ref[...],
                             preferred_element_type=jnp.float32) + b_ref[...]

    return pl.pallas_call(
        body,
        grid=(n // blk,),
        in_specs=[pl.BlockSpec((blk, d), lambda i: (i, 0)),
                  pl.BlockSpec((d, d), lambda i: (0, 0)),
                  pl.BlockSpec((1, d), lambda i: (0, 0))],
        out_specs=pl.BlockSpec((blk, d), lambda i: (i, 0)),
        out_shape=jax.ShapeDtypeStruct((n, d), jnp.float32),
    )(x, w, bvec)


def _tc_head2(a, b, w, bvec, blk):
    n, d = a.shape

    def body(a_ref, b2_ref, w_ref, b_ref, o_ref):
        o_ref[...] = jnp.dot(a_ref[...] + b2_ref[...], w_ref[...],
                             preferred_element_type=jnp.float32) + b_ref[...]

    return pl.pallas_call(
        body,
        grid=(n // blk,),
        in_specs=[pl.BlockSpec((blk, d), lambda i: (i, 0)),
                  pl.BlockSpec((blk, d), lambda i: (i, 0)),
                  pl.BlockSpec((d, d), lambda i: (0, 0)),
                  pl.BlockSpec((1, d), lambda i: (0, 0))],
        out_specs=pl.BlockSpec((blk, d), lambda i: (i, 0)),
        out_shape=jax.ShapeDtypeStruct((n, d), jnp.float32),
    )(a, b, w, bvec)


def _sc_scores(q, k, dst, src):
    n, d = q.shape
    h = d // _LN
    e = dst.shape[0]
    nch = e // _CH
    steps = (nch + _NW - 1) // _NW
    steps += steps % 2
    scale = 1.0 / math.sqrt(_LN)
    mesh = plsc.VectorSubcoreMesh(core_axis_name="c", subcore_axis_name="s")

    @functools.partial(
        pl.kernel,
        mesh=mesh,
        compiler_params=pltpu.CompilerParams(needs_layout_passes=False),
        out_type=jax.ShapeDtypeStruct((nch, h, _CH), jnp.float32),
        scratch_types=[
            pltpu.VMEM((_CH,), jnp.int32),
            pltpu.VMEM((_CH,), jnp.int32),
            pltpu.VMEM((_CH,), jnp.int32),
            pltpu.VMEM((_CH,), jnp.int32),
            pltpu.VMEM((_CH, d), jnp.float32),
            pltpu.VMEM((_CH, d), jnp.float32),
            pltpu.VMEM((_CH, d), jnp.float32),
            pltpu.VMEM((_CH, d), jnp.float32),
            pltpu.VMEM((h, _CH), jnp.float32),
            pltpu.SemaphoreType.DMA,
            pltpu.SemaphoreType.DMA,
        ],
    )
    def run(q_hbm, k_hbm, dst_hbm, src_hbm, out_hbm, idx_d0, idx_s0, idx_d1,
            idx_s1, qr0, kr0, qr1, kr1, sb, sem0, sem1):
        wid = lax.axis_index("s") * _NC + lax.axis_index("c")

        def issue(c, idx_d, idx_s, qr, kr, sem):
            @pl.when(c < nch)
            def _():
                base = c * _CH
                pltpu.sync_copy(dst_hbm.at[pl.ds(base, _CH)], idx_d)
                pltpu.sync_copy(src_hbm.at[pl.ds(base, _CH)], idx_s)
                pltpu.async_copy(q_hbm.at[idx_d], qr, sem)
                pltpu.async_copy(k_hbm.at[idx_s], kr, sem)

        def compute(c, idx_d, idx_s, qr, kr, sem):
            @pl.when(c < nch)
            def _():
                pltpu.make_async_copy(q_hbm.at[idx_d], qr, sem).wait()
                pltpu.make_async_copy(k_hbm.at[idx_s], kr, sem).wait()

                def group(gi, cc):
                    rot = lax.iota(jnp.int32, _LN)
                    rows = gi * _LN + rot
                    for hh in range(h):
                        acc = jnp.zeros((_LN,), jnp.float32)
                        for j in range(_LN):
                            col = hh * _LN + ((rot + j) & (_LN - 1))
                            qv = plsc.load_gather(qr, [rows, col])
                            kv = plsc.load_gather(kr, [rows, col])
                            acc = acc + qv * kv
                        sb[hh, pl.ds(gi * _LN, _LN)] = acc * scale
                    return cc

                lax.fori_loop(0, _CH // _LN, group, 0)
                pltpu.sync_copy(sb, out_hbm.at[c])

        issue(wid, idx_d0, idx_s0, qr0, kr0, sem0)

        def pair(p, carry):
            s0 = p * 2
            issue(wid + (s0 + 1) * _NW, idx_d1, idx_s1, qr1, kr1, sem1)
            compute(wid + s0 * _NW, idx_d0, idx_s0, qr0, kr0, sem0)
            issue(wid + (s0 + 2) * _NW, idx_d0, idx_s0, qr0, kr0, sem0)
            compute(wid + (s0 + 1) * _NW, idx_d1, idx_s1, qr1, kr1, sem1)
            return carry

        lax.fori_loop(0, steps // 2, pair, 0)

    return run(q, k, dst, src)


def _sc_aggregate(vv, scores, m16, iz16, dst, src):
    n, d = vv.shape
    h = d // _LN
    e = dst.shape[0]
    nch = e // _CH
    steps = (nch + _NW - 1) // _NW
    steps += steps % 2
    npad = -(-n // (_NS * _CH)) * (_NS * _CH)
    rows_sub = npad // _NS
    mesh = plsc.VectorSubcoreMesh(core_axis_name="c", subcore_axis_name="s")

    @functools.partial(
        pl.kernel,
        mesh=mesh,
        compiler_params=pltpu.CompilerParams(needs_layout_passes=False),
        out_type=jax.ShapeDtypeStruct((_NC, npad, d), jnp.float32),
        scratch_types=[
            pltpu.VMEM((_CH,), jnp.int32),
            pltpu.VMEM((_CH,), jnp.int32),
            pltpu.VMEM((_CH,), jnp.int32),
            pltpu.VMEM((_CH,), jnp.int32),
            pltpu.VMEM((_CH, d), jnp.float32),
            pltpu.VMEM((_CH, d), jnp.float32),
            pltpu.VMEM((h, _CH), jnp.float32),
            pltpu.VMEM((h, _CH), jnp.float32),
            pltpu.VMEM((_LN,), jnp.float32),
            pltpu.VMEM((_LN,), jnp.float32),
            pltpu.VMEM_SHARED((npad, d), jnp.float32),
            pltpu.SemaphoreType.DMA,
            pltpu.SemaphoreType.DMA,
        ],
    )
    def run(v_hbm, s_hbm, m_hbm, z_hbm, dst_hbm, src_hbm, out_hbm,
            idx_d0, idx_s0, idx_d1, idx_s1, vr0, vr1, sb0, sb1, mb, zb, acc,
            sem0, sem1):
        cid = lax.axis_index("c")
        sid = lax.axis_index("s")
        wid = sid * _NC + cid

        def zrow(r, cc):
            for hh in range(h):
                vr0[r, pl.ds(hh * _LN, _LN)] = jnp.zeros((_LN,), jnp.float32)
            return cc

        lax.fori_loop(0, _CH, zrow, 0)
        for j in range(rows_sub // _CH):
            pltpu.sync_copy(vr0.at[pl.ds(0, _CH)],
                            acc.at[pl.ds(sid * rows_sub + j * _CH, _CH)])
        plsc.subcore_barrier()
        pltpu.sync_copy(m_hbm, mb)
        pltpu.sync_copy(z_hbm, zb)

        def issue(c, idx_d, idx_s, vr, sb, sem):
            @pl.when(c < nch)
            def _():
                base = c * _CH
                pltpu.sync_copy(dst_hbm.at[pl.ds(base, _CH)], idx_d)
                pltpu.sync_copy(src_hbm.at[pl.ds(base, _CH)], idx_s)
                pltpu.async_copy(v_hbm.at[idx_s], vr, sem)
                pltpu.async_copy(s_hbm.at[c], sb, sem)

        def compute(c, idx_d, idx_s, vr, sb, sem):
            @pl.when(c < nch)
            def _():
                pltpu.make_async_copy(v_hbm.at[idx_s], vr, sem).wait()
                pltpu.make_async_copy(s_hbm.at[c], sb, sem).wait()
                mv = mb[...]
                zv = zb[...]

                def group(gi, cc):
                    gsl = pl.ds(gi * _LN, _LN)
                    wrows = [jnp.exp(sb[hh, gsl] - mv[hh]) * zv[hh]
                             for hh in range(h)]
                    for ee in range(_LN):
                        ei = gi * _LN + ee
                        for hh in range(h):
                            sl = pl.ds(hh * _LN, _LN)
                            vr[ei, sl] = vr[ei, sl] * wrows[hh][ee]
                    return cc

                lax.fori_loop(0, _CH // _LN, group, 0)
                pltpu.sync_copy(vr, acc.at[idx_d], add=True)

        issue(wid, idx_d0, idx_s0, vr0, sb0, sem0)

        def pair(p, carry):
            s0 = p * 2
            issue(wid + (s0 + 1) * _NW, idx_d1, idx_s1, vr1, sb1, sem1)
            compute(wid + s0 * _NW, idx_d0, idx_s0, vr0, sb0, sem0)
            issue(wid + (s0 + 2) * _NW, idx_d0, idx_s0, vr0, sb0, sem0)
            compute(wid + (s0 + 1) * _NW, idx_d1, idx_s1, vr1, sb1, sem1)
            return carry

        lax.fori_loop(0, steps // 2, pair, 0)
        plsc.subcore_barrier()
        for j in range(rows_sub // _CH):
            r0 = sid * rows_sub + j * _CH
            pltpu.sync_copy(acc.at[pl.ds(r0, _CH)],
                            out_hbm.at[cid, pl.ds(r0, _CH)])

    return run(vv, scores, m16, iz16, dst, src)


def kernel(vertex_ids, edge_ids, triangle_ids, edge_index, vertex_embed,
           edge_embed, triangle_embed, vertex_pos, edge_pos, triangle_pos,
           Wq, bq, Wk, bk, Wv, bv, Wo, bo, ln_g, ln_b, Wvh, bvh, Weh, beh,
           Wth, bth):
    n, d = vertex_embed.shape
    h = d // _LN
    nl = Wq.shape[0]
    src = edge_index[0]
    dst = edge_index[1]

    v = _tc_add(vertex_embed, vertex_pos, 1000)
    for i in range(nl):
        q, k, vv = _tc_qkv(v, Wq[i], bq[i].reshape(1, d), Wk[i],
                           bk[i].reshape(1, d), Wv[i], bv[i].reshape(1, d),
                           1000)
        scores = _sc_scores(q, k, dst, src)
        m = _tc_max(scores)
        iz = _tc_inv_sumexp(scores, m)
        m16 = jnp.pad(m.reshape(h), (0, _LN - h))
        iz16 = jnp.pad(iz.reshape(h), (0, _LN - h))
        out2 = _sc_aggregate(vv, scores, m16, iz16, dst, src)
        v = _tc_outproj(out2[0, :n], out2[1, :n], v, Wo[i], bo[i].reshape(1, d),
                        ln_g[i].reshape(1, d), ln_b[i].reshape(1, d), 1000)

    v_out = _tc_head(v, Wvh, bvh.reshape(1, d), 1000)
    e_out = _tc_head2(edge_embed, edge_pos, Weh, beh.reshape(1, d), 2000)
    t_out = _tc_head2(triangle_embed, triangle_pos, Wth, bth.reshape(1, d),
                      2000)
    return (v_out, e_out, t_out)


# speedup vs baseline: 8.4081x; 4.2727x over previous
import functools
import math

import jax
import jax.numpy as jnp
from jax import lax
from jax.experimental import pallas as pl
from jax.experimental.pallas import tpu as pltpu
from jax.experimental.pallas import tpu_sc as plsc

_NC = 2
_NS = 16
_NW = _NC * _NS
_CH = 128
_LN = 16


def _tc_add(a, b, blk):
    n, d = a.shape

    def body(a_ref, b_ref, o_ref):
        o_ref[...] = a_ref[...] + b_ref[...]

    return pl.pallas_call(
        body,
        grid=(n // blk,),
        in_specs=[pl.BlockSpec((blk, d), lambda i: (i, 0))] * 2,
        out_specs=pl.BlockSpec((blk, d), lambda i: (i, 0)),
        out_shape=jax.ShapeDtypeStruct((n, d), jnp.float32),
    )(a, b)


def _tc_qkv(x, wq, bq_, wk, bk_, wv, bv_, blk):
    n, d = x.shape

    def body(x_ref, wq_ref, bq_ref, wk_ref, bk_ref, wv_ref, bv_ref,
             q_ref, k_ref, v_ref):
        xv = x_ref[...]
        q_ref[...] = jnp.dot(xv, wq_ref[...],
                             preferred_element_type=jnp.float32) + bq_ref[...]
        k_ref[...] = jnp.dot(xv, wk_ref[...],
                             preferred_element_type=jnp.float32) + bk_ref[...]
        v_ref[...] = jnp.dot(xv, wv_ref[...],
                             preferred_element_type=jnp.float32) + bv_ref[...]

    wspec = pl.BlockSpec((d, d), lambda i: (0, 0))
    bspec = pl.BlockSpec((1, d), lambda i: (0, 0))
    xspec = pl.BlockSpec((blk, d), lambda i: (i, 0))
    oshape = jax.ShapeDtypeStruct((n, d), jnp.float32)
    return pl.pallas_call(
        body,
        grid=(n // blk,),
        in_specs=[xspec, wspec, bspec, wspec, bspec, wspec, bspec],
        out_specs=[xspec] * 3,
        out_shape=[oshape] * 3,
    )(x, wq, bq_, wk, bk_, wv, bv_)


def _tc_max(scores):
    nch, h, ch = scores.shape

    def body(s_ref, m_ref, acc):
        i = pl.program_id(0)

        @pl.when(i == 0)
        def _():
            acc[...] = s_ref[0]

        @pl.when(i > 0)
        def _():
            acc[...] = jnp.maximum(acc[...], s_ref[0])

        m_ref[...] = jnp.max(acc[...], axis=1, keepdims=True)

    return pl.pallas_call(
        body,
        grid=(nch,),
        in_specs=[pl.BlockSpec((1, h, ch), lambda i: (i, 0, 0))],
        out_specs=pl.BlockSpec((h, 1), lambda i: (0, 0)),
        out_shape=jax.ShapeDtypeStruct((h, 1), jnp.float32),
        scratch_shapes=[pltpu.VMEM((h, ch), jnp.float32)],
    )(scores)


def _tc_inv_sumexp(scores, m):
    nch, h, ch = scores.shape

    def body(s_ref, m_ref, z_ref, acc):
        i = pl.program_id(0)
        e = jnp.exp(s_ref[0] - m_ref[...])

        @pl.when(i == 0)
        def _():
            acc[...] = e

        @pl.when(i > 0)
        def _():
            acc[...] = acc[...] + e

        z_ref[...] = 1.0 / jnp.sum(acc[...], axis=1, keepdims=True)

    return pl.pallas_call(
        body,
        grid=(nch,),
        in_specs=[pl.BlockSpec((1, h, ch), lambda i: (i, 0, 0)),
                  pl.BlockSpec((h, 1), lambda i: (0, 0))],
        out_specs=pl.BlockSpec((h, 1), lambda i: (0, 0)),
        out_shape=jax.ShapeDtypeStruct((h, 1), jnp.float32),
        scratch_shapes=[pltpu.VMEM((h, ch), jnp.float32)],
    )(scores, m)


def _tc_outproj(o0, o1, zp, res, wo, bo_, g, b, blk):
    n, d = o0.shape
    nw = zp.shape[0]

    def body(o0_ref, o1_ref, zp_ref, r_ref, w_ref, b_ref, g_ref, be_ref,
             out_ref):
        z128 = jnp.sum(zp_ref[...], axis=0, keepdims=True)
        ri = lax.broadcasted_iota(jnp.int32, (d, d), 0) // _LN
        ci = lax.broadcasted_iota(jnp.int32, (d, d), 1) // _LN
        mask = (ri == ci).astype(jnp.float32)
        zt = jnp.dot(z128, mask, preferred_element_type=jnp.float32)
        o = (o0_ref[...] + o1_ref[...]) / zt
        y = jnp.dot(o, w_ref[...], preferred_element_type=jnp.float32)
        y = y + b_ref[...] + r_ref[...]
        mu = jnp.mean(y, axis=-1, keepdims=True)
        var = jnp.mean((y - mu) ** 2, axis=-1, keepdims=True)
        out_ref[...] = (y - mu) / jnp.sqrt(var + 1e-5) * g_ref[...] + be_ref[...]

    xspec = pl.BlockSpec((blk, d), lambda i: (i, 0))
    wspec = pl.BlockSpec((d, d), lambda i: (0, 0))
    bspec = pl.BlockSpec((1, d), lambda i: (0, 0))
    zspec = pl.BlockSpec((nw, d), lambda i: (0, 0))
    return pl.pallas_call(
        body,
        grid=(n // blk,),
        in_specs=[xspec, xspec, zspec, xspec, wspec, bspec, bspec, bspec],
        out_specs=xspec,
        out_shape=jax.ShapeDtypeStruct((n, d), jnp.float32),
    )(o0, o1, zp, res, wo, bo_, g, b)


def _tc_head(x, w, bvec, blk):
    n, d = x.shape

    def body(x_ref, w_ref, b_ref, o_ref):
        o_ref[...] = jnp.dot(x_ref[...], w_ref[...],
                             preferred_element_type=jnp.float32) + b_ref[...]

    return pl.pallas_call(
        body,
        grid=(n // blk,),
        in_specs=[pl.BlockSpec((blk, d), lambda i: (i, 0)),
                  pl.BlockSpec((d, d), lambda i: (0, 0)),
                  pl.BlockSpec((1, d), lambda i: (0, 0))],
        out_specs=pl.BlockSpec((blk, d), lambda i: (i, 0)),
        out_shape=jax.ShapeDtypeStruct((n, d), jnp.float32),
    )(x, w, bvec)


def _tc_head2(a, b, w, bvec, blk):
    n, d = a.shape

    def body(a_ref, b2_ref, w_ref, b_ref, o_ref):
        o_ref[...] = jnp.dot(a_ref[...] + b2_ref[...], w_ref[...],
                             preferred_element_type=jnp.float32) + b_ref[...]

    return pl.pallas_call(
        body,
        grid=(n // blk,),
        in_specs=[pl.BlockSpec((blk, d), lambda i: (i, 0)),
                  pl.BlockSpec((blk, d), lambda i: (i, 0)),
                  pl.BlockSpec((d, d), lambda i: (0, 0)),
                  pl.BlockSpec((1, d), lambda i: (0, 0))],
        out_specs=pl.BlockSpec((blk, d), lambda i: (i, 0)),
        out_shape=jax.ShapeDtypeStruct((n, d), jnp.float32),
    )(a, b, w, bvec)


def _sc_attention(q, k, vv, dst, src):
    n, d = q.shape
    h = d // _LN
    e = dst.shape[0]
    ch = 64
    ng = ch // _LN
    nch = e // ch
    steps = (nch + _NW - 1) // _NW
    steps += steps % 2
    rows_sub = -(-(n // _NS) // 8) * 8
    npad = rows_sub * _NS
    full, rem = divmod(rows_sub, ch)
    copy_plan = [(j * ch, ch) for j in range(full)]
    if rem:
        copy_plan.append((full * ch, rem))
    scale = 1.0 / math.sqrt(_LN)
    mesh = plsc.VectorSubcoreMesh(core_axis_name="c", subcore_axis_name="s")

    @functools.partial(
        pl.kernel,
        mesh=mesh,
        compiler_params=pltpu.CompilerParams(needs_layout_passes=False),
        out_type=(jax.ShapeDtypeStruct((_NC, npad, d), jnp.float32),
                  jax.ShapeDtypeStruct((_NW, h, _LN), jnp.float32)),
        scratch_types=[
            pltpu.VMEM((ch,), jnp.int32),
            pltpu.VMEM((ch,), jnp.int32),
            pltpu.VMEM((ch,), jnp.int32),
            pltpu.VMEM((ch,), jnp.int32),
            pltpu.VMEM((ch, d), jnp.float32),
            pltpu.VMEM((ch, d), jnp.float32),
            pltpu.VMEM((ch, d), jnp.float32),
            pltpu.VMEM((ch, d), jnp.float32),
            pltpu.VMEM((ch, d), jnp.float32),
            pltpu.VMEM((h, ch), jnp.float32),
            pltpu.VMEM((h, _LN), jnp.float32),
            pltpu.VMEM_SHARED((npad, d), jnp.float32),
            pltpu.SemaphoreType.DMA,
            pltpu.SemaphoreType.DMA,
            pltpu.SemaphoreType.DMA,
        ],
    )
    def run(q_hbm, k_hbm, v_hbm, dst_hbm, src_hbm, out_hbm, z_hbm,
            idx_d0, idx_s0, idx_d1, idx_s1, qr0, kr0, qr1, kr1, vr,
            wbuf, zbuf, acc, sem0, sem1, semv):
        cid = lax.axis_index("c")
        sid = lax.axis_index("s")
        wid = sid * _NC + cid

        def zrow(r, cc):
            for hh in range(h):
                vr[r, pl.ds(hh * _LN, _LN)] = jnp.zeros((_LN,), jnp.float32)
            return cc

        lax.fori_loop(0, ch, zrow, 0)
        for hh in range(h):
            zbuf[hh, :] = jnp.zeros((_LN,), jnp.float32)
        for off, sz in copy_plan:
            pltpu.sync_copy(vr.at[pl.ds(0, sz)],
                            acc.at[pl.ds(sid * rows_sub + off, sz)])
        plsc.subcore_barrier()

        def issue_qk(c, idx_d, idx_s, qr, kr, sem):
            @pl.when(c < nch)
            def _():
                base = c * ch
                pltpu.sync_copy(dst_hbm.at[pl.ds(base, ch)], idx_d)
                pltpu.sync_copy(src_hbm.at[pl.ds(base, ch)], idx_s)
                pltpu.async_copy(q_hbm.at[idx_d], qr, sem)
                pltpu.async_copy(k_hbm.at[idx_s], kr, sem)

        def issue_v(c, idx_s):
            @pl.when(c < nch)
            def _():
                pltpu.async_copy(v_hbm.at[idx_s], vr, semv)

        def process(c, nidx_s, idx_d, idx_s, qr, kr, sem):
            @pl.when(c < nch)
            def _():
                pltpu.make_async_copy(q_hbm.at[idx_d], qr, sem).wait()
                pltpu.make_async_copy(k_hbm.at[idx_s], kr, sem).wait()

                def group(gi, cc):
                    rot = lax.iota(jnp.int32, _LN)
                    rows = gi * _LN + rot
                    for hh in range(h):
                        acc_v = jnp.zeros((_LN,), jnp.float32)
                        for j in range(_LN):
                            col = hh * _LN + ((rot + j) & (_LN - 1))
                            qv = plsc.load_gather(qr, [rows, col])
                            kv = plsc.load_gather(kr, [rows, col])
                            acc_v = acc_v + qv * kv
                        w = jnp.exp(acc_v * scale)
                        wbuf[hh, pl.ds(gi * _LN, _LN)] = w
                        zbuf[hh, :] = zbuf[hh, :] + w
                    return cc

                lax.fori_loop(0, ng, group, 0)
                pltpu.make_async_copy(v_hbm.at[idx_s], vr, semv).wait()

                def sgroup(gi, cc):
                    gsl = pl.ds(gi * _LN, _LN)
                    wl = [wbuf[hh, gsl] for hh in range(h)]
                    for ee in range(_LN):
                        ei = gi * _LN + ee
                        for hh in range(h):
                            sl = pl.ds(hh * _LN, _LN)
                            vr[ei, sl] = vr[ei, sl] * wl[hh][ee]
                    return cc

                lax.fori_loop(0, ng, sgroup, 0)
                pltpu.sync_copy(vr, acc.at[idx_d], add=True)
                issue_v(c + _NW, nidx_s)

        issue_qk(wid, idx_d0, idx_s0, qr0, kr0, sem0)
        issue_v(wid, idx_s0)

        def pair(p, carry):
            s0 = p * 2
            issue_qk(wid + (s0 + 1) * _NW, idx_d1, idx_s1, qr1, kr1, sem1)
            process(wid + s0 * _NW, idx_s1, idx_d0, idx_s0, qr0, kr0, sem0)
            issue_qk(wid + (s0 + 2) * _NW, idx_d0, idx_s0, qr0, kr0, sem0)
            process(wid + (s0 + 1) * _NW, idx_s0, idx_d1, idx_s1, qr1, kr1,
                    sem1)
            return carry

        lax.fori_loop(0, steps // 2, pair, 0)
        pltpu.sync_copy(zbuf, z_hbm.at[wid])
        plsc.subcore_barrier()
        for off, sz in copy_plan:
            r0 = sid * rows_sub + off
            pltpu.sync_copy(acc.at[pl.ds(r0, sz)],
                            out_hbm.at[cid, pl.ds(r0, sz)])

    return run(q, k, vv, dst, src)


def kernel(vertex_ids, edge_ids, triangle_ids, edge_index, vertex_embed,
           edge_embed, triangle_embed, vertex_pos, edge_pos, triangle_pos,
           Wq, bq, Wk, bk, Wv, bv, Wo, bo, ln_g, ln_b, Wvh, bvh, Weh, beh,
           Wth, bth):
    n, d = vertex_embed.shape
    h = d // _LN
    nl = Wq.shape[0]
    src = edge_index[0]
    dst = edge_index[1]

    v = _tc_add(vertex_embed, vertex_pos, 1000)
    for i in range(nl):
        q, k, vv = _tc_qkv(v, Wq[i], bq[i].reshape(1, d), Wk[i],
                           bk[i].reshape(1, d), Wv[i], bv[i].reshape(1, d),
                           1000)
        out2, zp = _sc_attention(q, k, vv, dst, src)
        zp2 = zp.reshape(zp.shape[0], d)
        v = _tc_outproj(out2[0, :n], out2[1, :n], zp2, v, Wo[i],
                        bo[i].reshape(1, d),
                        ln_g[i].reshape(1, d), ln_b[i].reshape(1, d), 1000)

    v_out = _tc_head(v, Wvh, bvh.reshape(1, d), 1000)
    e_out = _tc_head2(edge_embed, edge_pos, Weh, beh.reshape(1, d), 2000)
    t_out = _tc_head2(triangle_embed, triangle_pos, Wth, bth.reshape(1, d),
                      2000)
    return (v_out, e_out, t_out)
